# baseline (speedup 1.0000x reference)
import jax
import jax.numpy as jnp
from jax import lax
from jax.experimental import pallas as pl
from jax.experimental.pallas import tpu as pltpu

_B = 1024
_NI = 512
_NO = 64
_H = 128

_OB = 16
_G = _NO // _OB


def _body(l1_ref, l2_ref, l3_ref, state_ref, opt_ref, out_ref):
    o = pl.program_id(0)
    l1b = l1_ref[...]
    l2b = l2_ref[...]
    l3b = l3_ref[...]
    u = lax.dot_general(l3b, l2b, (((1,), (2,)), ((0,), (0,))),
                        preferred_element_type=jnp.float32)
    vrow = lax.dot_general(u, l1b, (((2,), (2,)), ((0,), (0,))),
                           preferred_element_type=jnp.float32)
    vmat = vrow.reshape(_OB, _NI)
    part = lax.dot_general(state_ref[...], vmat, (((1,), (1,)), ((), ())),
                           preferred_element_type=jnp.float32)
    cols = o * _OB + lax.broadcasted_iota(jnp.int32, (1, _OB), 1)
    onehot = (opt_ref[...] == cols)
    contrib = jnp.sum(jnp.where(onehot, part, 0.0), axis=1, keepdims=True)

    @pl.when(o == 0)
    def _init():
        out_ref[...] = contrib

    @pl.when(o > 0)
    def _acc():
        out_ref[...] += contrib


def kernel(state, option, action, linear1, linear2, linear3):
    opt = option.astype(jnp.int32).reshape(_B, 1)
    out = pl.pallas_call(
        _body,
        grid=(_G,),
        in_specs=[
            pl.BlockSpec((_OB, _NI, _H), lambda o: (o, 0, 0)),
            pl.BlockSpec((_OB, _H, _H), lambda o: (o, 0, 0)),
            pl.BlockSpec((_OB, _H, 1), lambda o: (o, 0, 0)),
            pl.BlockSpec((_B, _NI), lambda o: (0, 0)),
            pl.BlockSpec((_B, 1), lambda o: (0, 0)),
        ],
        out_specs=pl.BlockSpec((_B, 1), lambda o: (0, 0)),
        out_shape=jax.ShapeDtypeStruct((_B, 1), jnp.float32),
    )(linear1, linear2, linear3, state, opt)
    return out
